# baseline (speedup 1.0000x reference)
import functools

import jax
import jax.numpy as jnp
from jax import lax
from jax.experimental import pallas as pl
from jax.experimental.pallas import tpu as pltpu
from jax.experimental.pallas import tpu_sc as plsc

_NUM_CORES = 2
_NUM_SUBCORES = 16
_NUM_WORKERS = _NUM_CORES * _NUM_SUBCORES
_LANES = 16

_CHUNKS = (104, 96)
_BLK = 128
_NBUF = 4


def _pool(inputs, bf16_table):
    B, CTX = inputs.shape
    V, d = bf16_table.shape
    assert d % (2 * _LANES) == 0
    groups = d // (2 * _LANES)
    assert sum(_CHUNKS) == CTX
    offs = [sum(_CHUNKS[:j]) for j in range(len(_CHUNKS))]
    assert B % _NUM_WORKERS == 0
    b_per_w = B // _NUM_WORKERS

    mesh = plsc.VectorSubcoreMesh(
        core_axis_name="c",
        subcore_axis_name="s",
        num_cores=_NUM_CORES,
        num_subcores=_NUM_SUBCORES,
    )

    @functools.partial(
        pl.kernel,
        out_type=jax.ShapeDtypeStruct((B, d), jnp.float32),
        mesh=mesh,
        scratch_types=[
            pltpu.VMEM((_BLK, CTX), jnp.int32),
            pltpu.VMEM((_NBUF, len(_CHUNKS), max(_CHUNKS), d),
                       jnp.bfloat16),
            pltpu.VMEM((_BLK, d), jnp.float32),
            [pltpu.SemaphoreType.DMA] * _NBUF,
        ],
        compiler_params=pltpu.CompilerParams(use_tc_tiling_on_sc=False,
                                             needs_layout_passes=False),
    )
    def pool_kernel(idx_hbm, table_hbm, out_hbm, idx_v, rows_v, pooled_v,
                    sems):
        wid = lax.axis_index("s") * _NUM_CORES + lax.axis_index("c")
        base = wid * b_per_w
        n_blk = b_per_w // _BLK

        def issue(g, p):
            for j, (o, n) in enumerate(zip(offs, _CHUNKS)):
                pltpu.async_copy(
                    table_hbm.at[idx_v.at[g, pl.ds(o, n)]],
                    rows_v.at[p, j, pl.ds(0, n)],
                    sems[p],
                )

        def drain(g, p):
            for j, (o, n) in enumerate(zip(offs, _CHUNKS)):
                pltpu.make_async_copy(
                    table_hbm.at[idx_v.at[g, pl.ds(o, n)]],
                    rows_v.at[p, j, pl.ds(0, n)],
                    sems[p],
                ).wait()

        hi_mask = jnp.int32(-65536)

        def accumulate(g, p):
            def body_for(j):
                def body(r, carry):
                    new = []
                    for c in range(groups):
                        w = plsc.bitcast(
                            rows_v[p, j, r, pl.ds(c * 2 * _LANES,
                                                  2 * _LANES)],
                            jnp.int32)
                        lo = plsc.bitcast(lax.shift_left(w, 16), jnp.float32)
                        hi = plsc.bitcast(lax.bitwise_and(w, hi_mask),
                                          jnp.float32)
                        new.append(carry[2 * c] + lo)
                        new.append(carry[2 * c + 1] + hi)
                    return tuple(new)
                return body

            acc = tuple(jnp.zeros((_LANES,), jnp.float32)
                        for _ in range(2 * groups))
            for j, n in enumerate(_CHUNKS):
                acc = lax.fori_loop(0, n, body_for(j), acc, unroll=4)
            for c in range(2 * groups):
                pooled_v[g, pl.ds(c * _LANES, _LANES)] = acc[c]

        assert _BLK % _NBUF == 0 and _BLK // _NBUF >= 2

        def block(k, carry):
            pltpu.sync_copy(idx_hbm.at[pl.ds(base + k * _BLK, _BLK)], idx_v)
            for p in range(_NBUF):
                issue(p, p)

            def step(i, carry):
                for p in range(_NBUF):
                    g = i * _NBUF + p
                    drain(g, p)
                    accumulate(g, p)
                    issue(g + _NBUF, p)
                return carry

            lax.fori_loop(0, _BLK // _NBUF - 1, step, 0)

            for p in range(_NBUF):
                g = _BLK - _NBUF + p
                drain(g, p)
                accumulate(g, p)

            pltpu.sync_copy(pooled_v,
                            out_hbm.at[pl.ds(base + k * _BLK, _BLK)])
            return carry

        lax.fori_loop(0, n_blk, block, 0)

    return pool_kernel(inputs, bf16_table)


def _linear(pooled, linear_w, linear_b, colmap):
    B, D = pooled.shape
    N = linear_w.shape[0]
    N_pad = (N + 127) // 128 * 128
    wt = jnp.pad(linear_w, ((0, N_pad - N), (0, 0))).T
    wt = wt[colmap, :]
    bias = jnp.pad(linear_b, (0, N_pad - N)).reshape(1, N_pad)

    BM = 1024

    def mm_body(x_ref, w_ref, b_ref, o_ref):
        o_ref[...] = (
            jnp.dot(x_ref[...], w_ref[...], preferred_element_type=jnp.float32)
            + b_ref[...]
        )

    out = pl.pallas_call(
        mm_body,
        grid=(B // BM,),
        in_specs=[
            pl.BlockSpec((BM, D), lambda i: (i, 0)),
            pl.BlockSpec((D, N_pad), lambda i: (0, 0)),
            pl.BlockSpec((1, N_pad), lambda i: (0, 0)),
        ],
        out_specs=pl.BlockSpec((BM, N_pad), lambda i: (i, 0)),
        out_shape=jax.ShapeDtypeStruct((B, N_pad), jnp.float32),
    )(pooled, wt, bias)
    return out[:, :N]


def kernel(inputs, embed_table, linear_w, linear_b):
    inputs = inputs.astype(jnp.int32)
    V, D = embed_table.shape
    assert D % 32 == 0
    table_bf16 = jax.lax.optimization_barrier(
        embed_table.astype(jnp.bfloat16))
    pooled = _pool(inputs, table_bf16)
    colmap = []
    for g in range(D // 32):
        colmap += [32 * g + 2 * k for k in range(16)]
        colmap += [32 * g + 2 * k + 1 for k in range(16)]
    return _linear(pooled, linear_w, linear_b, jnp.array(colmap))


# speedup vs baseline: 1.1028x; 1.1028x over previous
import functools

import jax
import jax.numpy as jnp
from jax import lax
from jax.experimental import pallas as pl
from jax.experimental.pallas import tpu as pltpu
from jax.experimental.pallas import tpu_sc as plsc

_NUM_CORES = 2
_NUM_SUBCORES = 16
_NUM_WORKERS = _NUM_CORES * _NUM_SUBCORES
_LANES = 16

_CHUNKS = (104, 96)
_BLK = 64
_NBUF = 8


def _pool(inputs, embed_table):
    B, CTX = inputs.shape
    V, D = embed_table.shape
    assert sum(_CHUNKS) == CTX
    offs = [sum(_CHUNKS[:j]) for j in range(len(_CHUNKS))]
    assert B % _NUM_WORKERS == 0
    b_per_w = B // _NUM_WORKERS
    assert D % _LANES == 0
    d_regs = D // _LANES

    mesh = plsc.VectorSubcoreMesh(
        core_axis_name="c",
        subcore_axis_name="s",
        num_cores=_NUM_CORES,
        num_subcores=_NUM_SUBCORES,
    )

    @functools.partial(
        pl.kernel,
        out_type=jax.ShapeDtypeStruct((B, D), jnp.float32),
        mesh=mesh,
        scratch_types=[
            pltpu.VMEM((_BLK, CTX), jnp.int32),
            pltpu.VMEM((_NBUF, CTX, D), jnp.float32),
            pltpu.VMEM((_BLK, D), jnp.float32),
            [pltpu.SemaphoreType.DMA] * _NBUF,
        ],
        compiler_params=pltpu.CompilerParams(use_tc_tiling_on_sc=False),
    )
    def pool_kernel(idx_hbm, table_hbm, out_hbm, idx_v, rows_v, pooled_v,
                    sems):
        wid = lax.axis_index("s") * _NUM_CORES + lax.axis_index("c")
        base = wid * b_per_w
        n_blk = b_per_w // _BLK

        def issue(g, p):
            for o, n in zip(offs, _CHUNKS):
                pltpu.async_copy(
                    table_hbm.at[idx_v.at[g, pl.ds(o, n)]],
                    rows_v.at[p, pl.ds(o, n)],
                    sems[p],
                )

        def drain(g, p):
            for o, n in zip(offs, _CHUNKS):
                pltpu.make_async_copy(
                    table_hbm.at[idx_v.at[g, pl.ds(o, n)]],
                    rows_v.at[p, pl.ds(o, n)],
                    sems[p],
                ).wait()

        def accumulate(g, p):
            def body(r, carry):
                return tuple(
                    carry[c] + rows_v[p, r, pl.ds(c * _LANES, _LANES)]
                    for c in range(d_regs)
                )
            acc = lax.fori_loop(
                0, CTX, body,
                tuple(jnp.zeros((_LANES,), jnp.float32) for _ in range(d_regs)),
                unroll=4,
            )
            for c in range(d_regs):
                pooled_v[g, pl.ds(c * _LANES, _LANES)] = acc[c]

        assert _BLK % _NBUF == 0 and _BLK // _NBUF >= 2

        def block(k, carry):
            pltpu.sync_copy(idx_hbm.at[pl.ds(base + k * _BLK, _BLK)], idx_v)
            for p in range(_NBUF):
                issue(p, p)

            def step(i, carry):
                for p in range(_NBUF):
                    g = i * _NBUF + p
                    drain(g, p)
                    accumulate(g, p)
                    issue(g + _NBUF, p)
                return carry

            lax.fori_loop(0, _BLK // _NBUF - 1, step, 0)

            for p in range(_NBUF):
                g = _BLK - _NBUF + p
                drain(g, p)
                accumulate(g, p)

            pltpu.sync_copy(pooled_v,
                            out_hbm.at[pl.ds(base + k * _BLK, _BLK)])
            return carry

        lax.fori_loop(0, n_blk, block, 0)

    return pool_kernel(inputs, embed_table)


def _linear(pooled, linear_w, linear_b):
    B, D = pooled.shape
    N = linear_w.shape[0]
    N_pad = (N + 127) // 128 * 128
    wt = jnp.pad(linear_w, ((0, N_pad - N), (0, 0))).T
    bias = jnp.pad(linear_b, (0, N_pad - N)).reshape(1, N_pad)

    BM = 1024

    def mm_body(x_ref, w_ref, b_ref, o_ref):
        o_ref[...] = (
            jnp.dot(x_ref[...], w_ref[...], preferred_element_type=jnp.float32)
            + b_ref[...]
        )

    out = pl.pallas_call(
        mm_body,
        grid=(B // BM,),
        in_specs=[
            pl.BlockSpec((BM, D), lambda i: (i, 0)),
            pl.BlockSpec((D, N_pad), lambda i: (0, 0)),
            pl.BlockSpec((1, N_pad), lambda i: (0, 0)),
        ],
        out_specs=pl.BlockSpec((BM, N_pad), lambda i: (i, 0)),
        out_shape=jax.ShapeDtypeStruct((B, N_pad), jnp.float32),
    )(pooled, wt, bias)
    return out[:, :N]


def kernel(inputs, embed_table, linear_w, linear_b):
    inputs = inputs.astype(jnp.int32)
    pooled = _pool(inputs, embed_table)
    return _linear(pooled, linear_w, linear_b)


# speedup vs baseline: 1.1176x; 1.0134x over previous
import functools

import jax
import jax.numpy as jnp
from jax import lax
from jax.experimental import pallas as pl
from jax.experimental.pallas import tpu as pltpu
from jax.experimental.pallas import tpu_sc as plsc

_NUM_CORES = 2
_NUM_SUBCORES = 16
_NUM_WORKERS = _NUM_CORES * _NUM_SUBCORES
_LANES = 16

_CHUNKS = (104, 96)
_BLK = 128
_NBUF = 4


def _pool(idx_flat, embed_table, B, CTX):
    V, D = embed_table.shape
    assert sum(_CHUNKS) == CTX
    offs = [sum(_CHUNKS[:j]) for j in range(len(_CHUNKS))]
    assert B % _NUM_WORKERS == 0
    b_per_w = B // _NUM_WORKERS
    assert D % _LANES == 0
    d_regs = D // _LANES

    mesh = plsc.VectorSubcoreMesh(
        core_axis_name="c",
        subcore_axis_name="s",
        num_cores=_NUM_CORES,
        num_subcores=_NUM_SUBCORES,
    )

    @functools.partial(
        pl.kernel,
        out_type=jax.ShapeDtypeStruct((B * D,), jnp.float32),
        mesh=mesh,
        scratch_types=[
            pltpu.VMEM((_BLK * CTX,), jnp.int32),
            pltpu.VMEM((_NBUF, CTX, D), jnp.float32),
            pltpu.VMEM((_BLK * D,), jnp.float32),
            [pltpu.SemaphoreType.DMA] * _NBUF,
        ],
        compiler_params=pltpu.CompilerParams(use_tc_tiling_on_sc=False),
    )
    def pool_kernel(idx_hbm, table_hbm, out_hbm, idx_v, rows_v, pooled_v,
                    sems):
        wid = lax.axis_index("s") * _NUM_CORES + lax.axis_index("c")
        base = wid * b_per_w
        n_blk = b_per_w // _BLK

        def issue(g, p):
            for o, n in zip(offs, _CHUNKS):
                pltpu.async_copy(
                    table_hbm.at[idx_v.at[pl.ds(g * CTX + o, n)]],
                    rows_v.at[p, pl.ds(o, n)],
                    sems[p],
                )

        def drain(g, p):
            for o, n in zip(offs, _CHUNKS):
                pltpu.make_async_copy(
                    table_hbm.at[idx_v.at[pl.ds(g * CTX + o, n)]],
                    rows_v.at[p, pl.ds(o, n)],
                    sems[p],
                ).wait()

        def accumulate(g, p):
            def body(r, carry):
                return tuple(
                    carry[c] + rows_v[p, r, pl.ds(c * _LANES, _LANES)]
                    for c in range(d_regs)
                )
            acc = lax.fori_loop(
                0, CTX, body,
                tuple(jnp.zeros((_LANES,), jnp.float32) for _ in range(d_regs)),
                unroll=4,
            )
            for c in range(d_regs):
                pooled_v[pl.ds(g * D + c * _LANES, _LANES)] = acc[c]

        assert _BLK % _NBUF == 0 and _BLK // _NBUF >= 2

        def block(k, carry):
            pltpu.sync_copy(
                idx_hbm.at[pl.ds((base + k * _BLK) * CTX, _BLK * CTX)], idx_v)
            for p in range(_NBUF):
                issue(p, p)

            def step(i, carry):
                for p in range(_NBUF):
                    g = i * _NBUF + p
                    drain(g, p)
                    accumulate(g, p)
                    issue(g + _NBUF, p)
                return carry

            lax.fori_loop(0, _BLK // _NBUF - 1, step, 0)

            for p in range(_NBUF):
                g = _BLK - _NBUF + p
                drain(g, p)
                accumulate(g, p)

            pltpu.sync_copy(
                pooled_v,
                out_hbm.at[pl.ds((base + k * _BLK) * D, _BLK * D)])
            return carry

        lax.fori_loop(0, n_blk, block, 0)

    return pool_kernel(idx_flat, embed_table).reshape(B, D)


def _linear(pooled, linear_w, linear_b):
    B, D = pooled.shape
    N = linear_w.shape[0]
    N_pad = (N + 127) // 128 * 128
    wt = jnp.pad(linear_w, ((0, N_pad - N), (0, 0))).T
    bias = jnp.pad(linear_b, (0, N_pad - N)).reshape(1, N_pad)

    BM = 1024

    def mm_body(x_ref, w_ref, b_ref, o_ref):
        o_ref[...] = (
            jnp.dot(x_ref[...], w_ref[...], preferred_element_type=jnp.float32)
            + b_ref[...]
        )

    out = pl.pallas_call(
        mm_body,
        grid=(B // BM,),
        in_specs=[
            pl.BlockSpec((BM, D), lambda i: (i, 0)),
            pl.BlockSpec((D, N_pad), lambda i: (0, 0)),
            pl.BlockSpec((1, N_pad), lambda i: (0, 0)),
        ],
        out_specs=pl.BlockSpec((BM, N_pad), lambda i: (i, 0)),
        out_shape=jax.ShapeDtypeStruct((B, N_pad), jnp.float32),
    )(pooled, wt, bias)
    return out[:, :N]


def kernel(inputs, embed_table, linear_w, linear_b):
    inputs = inputs.astype(jnp.int32)
    B, CTX = inputs.shape
    pooled = _pool(inputs.reshape(-1), embed_table, B, CTX)
    return _linear(pooled, linear_w, linear_b)
